# baseline (speedup 1.0000x reference)
import jax
import jax.numpy as jnp
from jax.experimental import pallas as pl
from jax.experimental.pallas import tpu as pltpu


def _gcn_kernel(adj_ref, x_ref, w1_ref, b1_ref, w2_ref, b2_ref,
                o_ref, s1_ref, h2_ref):
    p = pl.program_id(0)
    i = pl.program_id(1)
    bi = adj_ref.shape[0]

    @pl.when((p == 0) & (i == 0))
    def _():
        s1_ref[...] = jnp.dot(x_ref[...], w1_ref[...],
                              preferred_element_type=jnp.float32
                              ).astype(jnp.bfloat16)

    @pl.when(p == 0)
    def _():
        a16 = adj_ref[...].astype(jnp.bfloat16)
        h = jnp.dot(a16, s1_ref[...], preferred_element_type=jnp.float32)
        h = jnp.maximum(h + b1_ref[...], 0.0)
        h2_ref[pl.ds(i * bi, bi), :] = jnp.dot(
            h.astype(jnp.bfloat16), w2_ref[...].astype(jnp.bfloat16),
            preferred_element_type=jnp.float32)

    @pl.when(p == 1)
    def _():
        o = jnp.dot(adj_ref[...], h2_ref[...],
                    preferred_element_type=jnp.float32,
                    precision=jax.lax.Precision.DEFAULT) + b2_ref[...]
        m = jnp.max(o, axis=1, keepdims=True)
        lse = jnp.log(jnp.sum(jnp.exp(o - m), axis=1, keepdims=True)) + m
        o_ref[...] = o - lse


def kernel(x, adj, W1, b1, W2, b2):
    n, nfeat = x.shape
    nhid = W1.shape[1]
    nclass = W2.shape[1]
    b1r = b1.reshape(1, nhid)
    b2r = b2.reshape(1, nclass)

    bi = 400
    ni = n // bi
    out = pl.pallas_call(
        _gcn_kernel,
        grid=(2, ni),
        in_specs=[
            pl.BlockSpec((bi, n), lambda p, i: (i, 0)),
            pl.BlockSpec((n, nfeat), lambda p, i: (0, 0)),
            pl.BlockSpec((nfeat, nhid), lambda p, i: (0, 0)),
            pl.BlockSpec((1, nhid), lambda p, i: (0, 0)),
            pl.BlockSpec((nhid, nclass), lambda p, i: (0, 0)),
            pl.BlockSpec((1, nclass), lambda p, i: (0, 0)),
        ],
        out_specs=pl.BlockSpec((bi, nclass), lambda p, i: (p * i, 0)),
        out_shape=jax.ShapeDtypeStruct((n, nclass), jnp.float32),
        scratch_shapes=[
            pltpu.VMEM((n, nhid), jnp.bfloat16),
            pltpu.VMEM((n, nclass), jnp.float32),
        ],
    )(adj, x, W1, b1r, W2, b2r)

    return out


# speedup vs baseline: 1.0105x; 1.0105x over previous
import jax
import jax.numpy as jnp
from jax.experimental import pallas as pl
from jax.experimental.pallas import tpu as pltpu


def _gcn_kernel(adj_ref, x_ref, w1_ref, b1_ref, w2_ref, b2_ref,
                o_ref, s1_ref, h2_ref):
    p = pl.program_id(0)
    iraw = pl.program_id(1)
    ni = pl.num_programs(1)
    i = jnp.where(p == 1, ni - 1 - iraw, iraw)
    bi = adj_ref.shape[0]

    @pl.when((p == 0) & (i == 0))
    def _():
        s1_ref[...] = jnp.dot(x_ref[...], w1_ref[...],
                              preferred_element_type=jnp.float32
                              ).astype(jnp.bfloat16)

    @pl.when(p == 0)
    def _():
        a16 = adj_ref[...].astype(jnp.bfloat16)
        h = jnp.dot(a16, s1_ref[...], preferred_element_type=jnp.float32)
        h = jnp.maximum(h + b1_ref[...], 0.0)
        h2_ref[pl.ds(i * bi, bi), :] = jnp.dot(
            h.astype(jnp.bfloat16), w2_ref[...].astype(jnp.bfloat16),
            preferred_element_type=jnp.float32)

    @pl.when(p == 1)
    def _():
        o = jnp.dot(adj_ref[...], h2_ref[...],
                    preferred_element_type=jnp.float32,
                    precision=jax.lax.Precision.DEFAULT) + b2_ref[...]
        m = jnp.max(o, axis=1, keepdims=True)
        lse = jnp.log(jnp.sum(jnp.exp(o - m), axis=1, keepdims=True)) + m
        o_ref[...] = o - lse


def kernel(x, adj, W1, b1, W2, b2):
    n, nfeat = x.shape
    nhid = W1.shape[1]
    nclass = W2.shape[1]
    b1r = b1.reshape(1, nhid)
    b2r = b2.reshape(1, nclass)

    bi = 400
    ni = n // bi
    out = pl.pallas_call(
        _gcn_kernel,
        grid=(2, ni),
        in_specs=[
            pl.BlockSpec((bi, n),
                         lambda p, i: (i + p * (ni - 1 - 2 * i), 0)),
            pl.BlockSpec((n, nfeat), lambda p, i: (0, 0)),
            pl.BlockSpec((nfeat, nhid), lambda p, i: (0, 0)),
            pl.BlockSpec((1, nhid), lambda p, i: (0, 0)),
            pl.BlockSpec((nhid, nclass), lambda p, i: (0, 0)),
            pl.BlockSpec((1, nclass), lambda p, i: (0, 0)),
        ],
        out_specs=pl.BlockSpec((bi, nclass),
                               lambda p, i: (p * (ni - 1 - i), 0)),
        out_shape=jax.ShapeDtypeStruct((n, nclass), jnp.float32),
        scratch_shapes=[
            pltpu.VMEM((n, nhid), jnp.bfloat16),
            pltpu.VMEM((n, nclass), jnp.float32),
        ],
    )(adj, x, W1, b1r, W2, b2r)

    return out
